# Initial kernel scaffold; baseline (speedup 1.0000x reference)
import functools

import jax
import jax.numpy as jnp
from jax import lax
from jax.experimental import pallas as pl
from jax.experimental.pallas import tpu as pltpu
from jax.experimental.pallas import tpu_sc as plsc

N = 10000
E = 320000
F = 128
C = 64

NC = 2
NS = 16
NW = NC * NS
CHUNK = 128
CH = 80
E_PAD = NW * CH * CHUNK
N_PAD = 10240
ROWS_PER_TILE = N_PAD // NS
INIT_CHUNKS = ROWS_PER_TILE // CHUNK


def _sc_mesh():
    return plsc.VectorSubcoreMesh(core_axis_name="c", subcore_axis_name="s")


def _sc_degree(dst3):
    @functools.partial(
        pl.kernel,
        out_type=jax.ShapeDtypeStruct((NC, N_PAD), jnp.float32),
        mesh=_sc_mesh(),
        scratch_types=[
            pltpu.VMEM((CH, CHUNK), jnp.int32),
            pltpu.VMEM((CHUNK,), jnp.float32),
            pltpu.VMEM((ROWS_PER_TILE,), jnp.float32),
            pltpu.VMEM_SHARED((N_PAD,), jnp.float32),
        ],
    )
    def deg_kernel(dst_hbm, out_hbm, dst_v, ones_v, buf_v, deg_sh):
        cid = lax.axis_index("c")
        sid = lax.axis_index("s")
        wid = cid * NS + sid

        def fill(i, _):
            buf_v[pl.ds(i * 16, 16)] = jnp.zeros((16,), jnp.float32)
            return 0

        lax.fori_loop(0, ROWS_PER_TILE // 16, fill, 0)

        def fill1(i, _):
            ones_v[pl.ds(i * 16, 16)] = jnp.ones((16,), jnp.float32)
            return 0

        lax.fori_loop(0, CHUNK // 16, fill1, 0)

        tile_rows = pl.ds(sid * ROWS_PER_TILE, ROWS_PER_TILE)
        pltpu.sync_copy(buf_v, deg_sh.at[tile_rows])
        pltpu.sync_copy(dst_hbm.at[wid], dst_v)
        plsc.subcore_barrier()

        def body(j, _):
            pltpu.sync_copy(ones_v, deg_sh.at[dst_v.at[j]], add=True)
            return 0

        lax.fori_loop(0, CH, body, 0)
        plsc.subcore_barrier()

        pltpu.sync_copy(deg_sh.at[tile_rows], buf_v)
        pltpu.sync_copy(buf_v, out_hbm.at[cid, tile_rows])

    return deg_kernel(dst3)


def _sc_aggregate(y_ext, src3, dst3):
    @functools.partial(
        pl.kernel,
        out_type=jax.ShapeDtypeStruct((NC, N_PAD, C), jnp.float32),
        mesh=_sc_mesh(),
        scratch_types=[
            pltpu.VMEM((CH, CHUNK), jnp.int32),
            pltpu.VMEM((CH, CHUNK), jnp.int32),
            pltpu.VMEM((CHUNK, C), jnp.float32),
            pltpu.VMEM((CHUNK, C), jnp.float32),
            pltpu.VMEM_SHARED((N_PAD, C), jnp.float32),
            pltpu.SemaphoreType.DMA,
            pltpu.SemaphoreType.DMA,
        ],
    )
    def agg_kernel(y_hbm, src_hbm, dst_hbm, out_hbm,
                   src_v, dst_v, rows_a, rows_b, agg_sh, sem_a, sem_b):
        cid = lax.axis_index("c")
        sid = lax.axis_index("s")
        wid = cid * NS + sid

        def zero_row(i, _):
            for k in range(C // 16):
                rows_b[i, pl.ds(k * 16, 16)] = jnp.zeros((16,), jnp.float32)
            return 0

        lax.fori_loop(0, CHUNK, zero_row, 0)

        for k in range(INIT_CHUNKS):
            sl = pl.ds(sid * ROWS_PER_TILE + k * CHUNK, CHUNK)

            @pl.when(cid == 0)
            def _():
                pltpu.sync_copy(y_hbm.at[sl], rows_a)
                pltpu.sync_copy(rows_a, agg_sh.at[sl])

            @pl.when(cid != 0)
            def _():
                pltpu.sync_copy(rows_b, agg_sh.at[sl])

        pltpu.sync_copy(src_hbm.at[wid], src_v)
        pltpu.sync_copy(dst_hbm.at[wid], dst_v)
        plsc.subcore_barrier()

        pltpu.async_copy(y_hbm.at[src_v.at[0]], rows_a, sem_a)
        pltpu.async_copy(y_hbm.at[src_v.at[1]], rows_b, sem_b)

        def body(jj, _):
            j = jj * 2
            pltpu.make_async_copy(y_hbm.at[src_v.at[j]], rows_a, sem_a).wait()
            pltpu.sync_copy(rows_a, agg_sh.at[dst_v.at[j]], add=True)
            pltpu.async_copy(y_hbm.at[src_v.at[j + 2]], rows_a, sem_a)
            pltpu.make_async_copy(y_hbm.at[src_v.at[j + 1]], rows_b, sem_b).wait()
            pltpu.sync_copy(rows_b, agg_sh.at[dst_v.at[j + 1]], add=True)
            pltpu.async_copy(y_hbm.at[src_v.at[j + 3]], rows_b, sem_b)
            return 0

        lax.fori_loop(0, CH // 2 - 1, body, 0)
        j = CH - 2
        pltpu.make_async_copy(y_hbm.at[src_v.at[j]], rows_a, sem_a).wait()
        pltpu.sync_copy(rows_a, agg_sh.at[dst_v.at[j]], add=True)
        pltpu.make_async_copy(y_hbm.at[src_v.at[j + 1]], rows_b, sem_b).wait()
        pltpu.sync_copy(rows_b, agg_sh.at[dst_v.at[j + 1]], add=True)
        plsc.subcore_barrier()

        for k in range(INIT_CHUNKS):
            sl = pl.ds(sid * ROWS_PER_TILE + k * CHUNK, CHUNK)
            pltpu.sync_copy(agg_sh.at[sl], rows_a)
            pltpu.sync_copy(rows_a, out_hbm.at[cid, sl])

    return agg_kernel(y_ext, src3, dst3)


def _tc_scale(x_pad, W, degp_t):
    BLK = 1024

    def body(x_ref, w_ref, d_ref, y_ref, dv_ref):
        deg = d_ref[:, 0] + d_ref[:, 1] + 1.0
        dinv = lax.rsqrt(deg)
        xw = jnp.dot(x_ref[...], w_ref[...], preferred_element_type=jnp.float32)
        y_ref[...] = xw * dinv[:, None]
        dv_ref[...] = jnp.broadcast_to(dinv[:, None], (BLK, C))

    return pl.pallas_call(
        body,
        grid=(N_PAD // BLK,),
        in_specs=[
            pl.BlockSpec((BLK, F), lambda i: (i, 0)),
            pl.BlockSpec((F, C), lambda i: (0, 0)),
            pl.BlockSpec((BLK, NC), lambda i: (i, 0)),
        ],
        out_specs=[
            pl.BlockSpec((BLK, C), lambda i: (i, 0)),
            pl.BlockSpec((BLK, C), lambda i: (i, 0)),
        ],
        out_shape=[
            jax.ShapeDtypeStruct((N_PAD, C), jnp.float32),
            jax.ShapeDtypeStruct((N_PAD, C), jnp.float32),
        ],
    )(x_pad, W, degp_t)


def _tc_final(aggp, dinvb, b2):
    BLK = 1000

    def body(a_ref, dv_ref, b_ref, o_ref):
        s = a_ref[0] + a_ref[1]
        o = s * dv_ref[...] + b_ref[...]
        m = jnp.max(o, axis=1, keepdims=True)
        ex = jnp.exp(o - m)
        ssum = jnp.sum(ex, axis=1, keepdims=True)
        o_ref[...] = (o - m) - jnp.log(ssum)

    return pl.pallas_call(
        body,
        grid=(N // BLK,),
        in_specs=[
            pl.BlockSpec((NC, BLK, C), lambda i: (0, i, 0)),
            pl.BlockSpec((BLK, C), lambda i: (i, 0)),
            pl.BlockSpec((1, C), lambda i: (0, 0)),
        ],
        out_specs=pl.BlockSpec((BLK, C), lambda i: (i, 0)),
        out_shape=jax.ShapeDtypeStruct((N, C), jnp.float32),
    )(aggp, dinvb, b2)


def kernel(x, edge_index, W, b):
    src = edge_index[0].astype(jnp.int32)
    dst = edge_index[1].astype(jnp.int32)
    npad = E_PAD - E
    pad_idx = N + (jnp.arange(npad, dtype=jnp.int32) % (N_PAD - N))
    src3 = jnp.concatenate([src, pad_idx]).reshape(NW, CH, CHUNK)
    dst3 = jnp.concatenate([dst, pad_idx]).reshape(NW, CH, CHUNK)

    degp = _sc_degree(dst3)
    x_pad = jnp.pad(x, ((0, N_PAD - N), (0, 0)))
    y_ext, dinvb = _tc_scale(x_pad, W, degp.T)
    aggp = _sc_aggregate(y_ext, src3, dst3)
    return _tc_final(aggp, dinvb, b.reshape(1, C))


# speedup vs baseline: 48.6919x; 48.6919x over previous
import functools

import jax
import jax.numpy as jnp
from jax import lax
from jax.experimental import pallas as pl
from jax.experimental.pallas import tpu as pltpu
from jax.experimental.pallas import tpu_sc as plsc

N = 10000
E = 320000
F = 128
C = 64

NC = 2
NS = 16
NW = NC * NS
CHUNK = 128
CH = 80
E_PAD = NW * CH * CHUNK
N_PAD = 10240
ROWS_PER_TILE = N_PAD // NS
INIT_CHUNKS = ROWS_PER_TILE // CHUNK


def _sc_mesh():
    return plsc.VectorSubcoreMesh(core_axis_name="c", subcore_axis_name="s")


_SC_PARAMS = pltpu.CompilerParams(use_tc_tiling_on_sc=False)


def _sc_degree(dst3):
    @functools.partial(
        pl.kernel,
        out_type=jax.ShapeDtypeStruct((NC, N_PAD), jnp.float32),
        mesh=_sc_mesh(),
        compiler_params=_SC_PARAMS,
        scratch_types=[
            pltpu.VMEM((CH, CHUNK), jnp.int32),
            pltpu.VMEM((CHUNK,), jnp.float32),
            pltpu.VMEM((ROWS_PER_TILE,), jnp.float32),
            pltpu.VMEM_SHARED((N_PAD,), jnp.float32),
        ],
    )
    def deg_kernel(dst_hbm, out_hbm, dst_v, ones_v, buf_v, deg_sh):
        cid = lax.axis_index("c")
        sid = lax.axis_index("s")
        wid = cid * NS + sid

        def fill(i, _):
            buf_v[pl.ds(i * 16, 16)] = jnp.zeros((16,), jnp.float32)
            return 0

        lax.fori_loop(0, ROWS_PER_TILE // 16, fill, 0)

        def fill1(i, _):
            ones_v[pl.ds(i * 16, 16)] = jnp.ones((16,), jnp.float32)
            return 0

        lax.fori_loop(0, CHUNK // 16, fill1, 0)

        tile_rows = pl.ds(sid * ROWS_PER_TILE, ROWS_PER_TILE)
        pltpu.sync_copy(buf_v, deg_sh.at[tile_rows])
        pltpu.sync_copy(dst_hbm.at[wid], dst_v)
        plsc.subcore_barrier()

        def body(j, _):
            pltpu.sync_copy(ones_v, deg_sh.at[dst_v.at[j]], add=True)
            return 0

        lax.fori_loop(0, CH, body, 0)
        plsc.subcore_barrier()

        pltpu.sync_copy(deg_sh.at[tile_rows], buf_v)
        pltpu.sync_copy(buf_v, out_hbm.at[cid, tile_rows])

    return deg_kernel(dst3)


def _sc_aggregate(y_ext, src3, dst3):
    @functools.partial(
        pl.kernel,
        out_type=jax.ShapeDtypeStruct((NC, N_PAD, C), jnp.float32),
        mesh=_sc_mesh(),
        compiler_params=_SC_PARAMS,
        scratch_types=[
            pltpu.VMEM((CH, CHUNK), jnp.int32),
            pltpu.VMEM((CH, CHUNK), jnp.int32),
            pltpu.VMEM((CHUNK, C), jnp.float32),
            pltpu.VMEM((CHUNK, C), jnp.float32),
            pltpu.VMEM_SHARED((N_PAD, C), jnp.float32),
            pltpu.SemaphoreType.DMA,
            pltpu.SemaphoreType.DMA,
        ],
    )
    def agg_kernel(y_hbm, src_hbm, dst_hbm, out_hbm,
                   src_v, dst_v, rows_a, rows_b, agg_sh, sem_a, sem_b):
        cid = lax.axis_index("c")
        sid = lax.axis_index("s")
        wid = cid * NS + sid

        def zero_row(i, _):
            for k in range(C // 16):
                rows_b[i, pl.ds(k * 16, 16)] = jnp.zeros((16,), jnp.float32)
            return 0

        lax.fori_loop(0, CHUNK, zero_row, 0)

        for k in range(INIT_CHUNKS):
            sl = pl.ds(sid * ROWS_PER_TILE + k * CHUNK, CHUNK)

            @pl.when(cid == 0)
            def _():
                pltpu.sync_copy(y_hbm.at[sl], rows_a)
                pltpu.sync_copy(rows_a, agg_sh.at[sl])

            @pl.when(cid != 0)
            def _():
                pltpu.sync_copy(rows_b, agg_sh.at[sl])

        pltpu.sync_copy(src_hbm.at[wid], src_v)
        pltpu.sync_copy(dst_hbm.at[wid], dst_v)
        plsc.subcore_barrier()

        pltpu.async_copy(y_hbm.at[src_v.at[0]], rows_a, sem_a)
        pltpu.async_copy(y_hbm.at[src_v.at[1]], rows_b, sem_b)

        def body(jj, _):
            j = jj * 2
            pltpu.make_async_copy(y_hbm.at[src_v.at[j]], rows_a, sem_a).wait()
            pltpu.sync_copy(rows_a, agg_sh.at[dst_v.at[j]], add=True)
            pltpu.async_copy(y_hbm.at[src_v.at[j + 2]], rows_a, sem_a)
            pltpu.make_async_copy(y_hbm.at[src_v.at[j + 1]], rows_b, sem_b).wait()
            pltpu.sync_copy(rows_b, agg_sh.at[dst_v.at[j + 1]], add=True)
            pltpu.async_copy(y_hbm.at[src_v.at[j + 3]], rows_b, sem_b)
            return 0

        lax.fori_loop(0, CH // 2 - 1, body, 0)
        j = CH - 2
        pltpu.make_async_copy(y_hbm.at[src_v.at[j]], rows_a, sem_a).wait()
        pltpu.sync_copy(rows_a, agg_sh.at[dst_v.at[j]], add=True)
        pltpu.make_async_copy(y_hbm.at[src_v.at[j + 1]], rows_b, sem_b).wait()
        pltpu.sync_copy(rows_b, agg_sh.at[dst_v.at[j + 1]], add=True)
        plsc.subcore_barrier()

        for k in range(INIT_CHUNKS):
            sl = pl.ds(sid * ROWS_PER_TILE + k * CHUNK, CHUNK)
            pltpu.sync_copy(agg_sh.at[sl], rows_a)
            pltpu.sync_copy(rows_a, out_hbm.at[cid, sl])

    return agg_kernel(y_ext, src3, dst3)


def _tc_scale(x_pad, W, degp_t):
    BLK = 1024

    def body(x_ref, w_ref, d_ref, y_ref, dv_ref):
        deg = d_ref[:, 0] + d_ref[:, 1] + 1.0
        dinv = lax.rsqrt(deg)
        xw = jnp.dot(x_ref[...], w_ref[...], preferred_element_type=jnp.float32)
        y_ref[...] = xw * dinv[:, None]
        dv_ref[...] = jnp.broadcast_to(dinv[:, None], (BLK, C))

    return pl.pallas_call(
        body,
        grid=(N_PAD // BLK,),
        in_specs=[
            pl.BlockSpec((BLK, F), lambda i: (i, 0)),
            pl.BlockSpec((F, C), lambda i: (0, 0)),
            pl.BlockSpec((BLK, NC), lambda i: (i, 0)),
        ],
        out_specs=[
            pl.BlockSpec((BLK, C), lambda i: (i, 0)),
            pl.BlockSpec((BLK, C), lambda i: (i, 0)),
        ],
        out_shape=[
            jax.ShapeDtypeStruct((N_PAD, C), jnp.float32),
            jax.ShapeDtypeStruct((N_PAD, C), jnp.float32),
        ],
    )(x_pad, W, degp_t)


def _tc_final(aggp, dinvb, b2):
    BLK = 1000

    def body(a_ref, dv_ref, b_ref, o_ref):
        s = a_ref[0] + a_ref[1]
        o = s * dv_ref[...] + b_ref[...]
        m = jnp.max(o, axis=1, keepdims=True)
        ex = jnp.exp(o - m)
        ssum = jnp.sum(ex, axis=1, keepdims=True)
        o_ref[...] = (o - m) - jnp.log(ssum)

    return pl.pallas_call(
        body,
        grid=(N // BLK,),
        in_specs=[
            pl.BlockSpec((NC, BLK, C), lambda i: (0, i, 0)),
            pl.BlockSpec((BLK, C), lambda i: (i, 0)),
            pl.BlockSpec((1, C), lambda i: (0, 0)),
        ],
        out_specs=pl.BlockSpec((BLK, C), lambda i: (i, 0)),
        out_shape=jax.ShapeDtypeStruct((N, C), jnp.float32),
    )(aggp, dinvb, b2)


def kernel(x, edge_index, W, b):
    src = edge_index[0].astype(jnp.int32)
    dst = edge_index[1].astype(jnp.int32)
    npad = E_PAD - E
    pad_idx = N + (jnp.arange(npad, dtype=jnp.int32) % (N_PAD - N))
    src3 = jnp.concatenate([src, pad_idx]).reshape(NW, CH, CHUNK)
    dst3 = jnp.concatenate([dst, pad_idx]).reshape(NW, CH, CHUNK)

    degp = _sc_degree(dst3)
    x_pad = jnp.pad(x, ((0, N_PAD - N), (0, 0)))
    y_ext, dinvb = _tc_scale(x_pad, W, degp.T)
    aggp = _sc_aggregate(y_ext, src3, dst3)
    return _tc_final(aggp, dinvb, b.reshape(1, C))
